# baseline (speedup 1.0000x reference)
import functools

import jax
import jax.numpy as jnp
from jax.experimental import pallas as pl

_E_PAD = 128
_BT = 512


def _gate_body(x_ref, wt_ref, bias_ref, idx_ref, score_ref):
    x = x_ref[...]
    logits = jnp.dot(x, wt_ref[...], preferred_element_type=jnp.float32)
    logits = logits + bias_ref[...]
    m = jnp.max(logits, axis=1, keepdims=True)
    e = jnp.exp(logits - m)
    s = jnp.sum(e, axis=1, keepdims=True)
    sm = e / s
    v = jnp.max(sm, axis=1, keepdims=True)
    lane = jax.lax.broadcasted_iota(jnp.int32, sm.shape, 1)
    idx = jnp.min(jnp.where(sm >= v, lane, _E_PAD), axis=1, keepdims=True)
    idx_ref[...] = idx
    score_ref[...] = v


@functools.partial(jax.jit, static_argnames=())
def kernel(inp, W, b):
    T, D = inp.shape
    E = W.shape[0]
    wt = jnp.zeros((D, _E_PAD), dtype=jnp.float32).at[:, :E].set(W.T)
    bias = jnp.full((1, _E_PAD), -1e30, dtype=jnp.float32).at[0, :E].set(b)
    grid = (T // _BT,)
    idx, score = pl.pallas_call(
        _gate_body,
        grid=grid,
        in_specs=[
            pl.BlockSpec((_BT, D), lambda i: (i, 0)),
            pl.BlockSpec((D, _E_PAD), lambda i: (0, 0)),
            pl.BlockSpec((1, _E_PAD), lambda i: (0, 0)),
        ],
        out_specs=[
            pl.BlockSpec((_BT, 1), lambda i: (i, 0)),
            pl.BlockSpec((_BT, 1), lambda i: (i, 0)),
        ],
        out_shape=[
            jax.ShapeDtypeStruct((T, 1), jnp.int32),
            jax.ShapeDtypeStruct((T, 1), jnp.float32),
        ],
    )(inp, wt, bias)
    return (idx.astype(jnp.int64), score)


# speedup vs baseline: 1.2622x; 1.2622x over previous
import functools

import jax
import jax.numpy as jnp
from jax.experimental import pallas as pl

_E_PAD = 128
_BT = 2048


def _gate_body(x_ref, wt_ref, bias_ref, idx_ref, score_ref):
    x = x_ref[...]
    logits = jnp.dot(x, wt_ref[...], preferred_element_type=jnp.float32)
    logits = logits + bias_ref[...]
    m = jnp.max(logits, axis=1, keepdims=True)
    e = jnp.exp(logits - m)
    s = jnp.sum(e, axis=1, keepdims=True)
    sm = e / s
    v = jnp.max(sm, axis=1, keepdims=True)
    lane = jax.lax.broadcasted_iota(jnp.int32, sm.shape, 1)
    idx = jnp.min(jnp.where(sm >= v, lane, _E_PAD), axis=1, keepdims=True)
    idx_ref[...] = idx
    score_ref[...] = v


@functools.partial(jax.jit, static_argnames=())
def kernel(inp, W, b):
    T, D = inp.shape
    E = W.shape[0]
    wt = jnp.zeros((D, _E_PAD), dtype=jnp.float32).at[:, :E].set(W.T)
    bias = jnp.full((1, _E_PAD), -1e30, dtype=jnp.float32).at[0, :E].set(b)
    grid = (T // _BT,)
    idx, score = pl.pallas_call(
        _gate_body,
        grid=grid,
        in_specs=[
            pl.BlockSpec((_BT, D), lambda i: (i, 0)),
            pl.BlockSpec((D, _E_PAD), lambda i: (0, 0)),
            pl.BlockSpec((1, _E_PAD), lambda i: (0, 0)),
        ],
        out_specs=[
            pl.BlockSpec((_BT, 1), lambda i: (i, 0)),
            pl.BlockSpec((_BT, 1), lambda i: (i, 0)),
        ],
        out_shape=[
            jax.ShapeDtypeStruct((T, 1), jnp.int32),
            jax.ShapeDtypeStruct((T, 1), jnp.float32),
        ],
    )(inp, wt, bias)
    return (idx.astype(jnp.int64), score)
